# baseline (speedup 1.0000x reference)
import functools
import jax
import jax.numpy as jnp
from jax import lax
from jax.experimental import pallas as pl
from jax.experimental.pallas import tpu as pltpu
from jax.experimental.pallas import tpu_sc as plsc

N = 10000
E = 160000
C = 128
A = 10
R = 8
LDIMS = [1, 3, 5]
D = 9
NPATH = 3
H = 64

NUM_TILES = 32
NODES_PER_TILE = 80
NODES_PER_PASS = NUM_TILES * NODES_PER_TILE
NPASS = 4
N_PAD = NPASS * NODES_PER_PASS

CHUNK = 800
NCHUNK = E // CHUNK
VPC = CHUNK // 16
MBUF = CHUNK + 64
GB = 32
ROW = D * C
WROW = (NPATH + 1) * C

_LIDX = []
for _li, _d in enumerate(LDIMS):
    _LIDX += [_li] * _d


def _silu(x):
    return x * jax.nn.sigmoid(x)


def _pre_h_body(nf_ref, wp_ref, h_ref):
    h_ref[...] = jnp.dot(nf_ref[...], wp_ref[...],
                         preferred_element_type=jnp.float32)


def _pre_h(nf, W_pre):
    BN = 1000
    return pl.pallas_call(
        _pre_h_body,
        grid=(N // BN,),
        in_specs=[
            pl.BlockSpec((BN, C), lambda i: (i, 0)),
            pl.BlockSpec((C, C), lambda i: (0, 0)),
        ],
        out_specs=pl.BlockSpec((BN, C), lambda i: (i, 0)),
        out_shape=jax.ShapeDtypeStruct((N, C), jnp.float32),
    )(nf, W_pre)


def _pre_w_body(rb_ref, sph_ref, w1_ref, w2_ref, w3_ref, w_ref):
    x = _silu(jnp.dot(rb_ref[...], w1_ref[...],
                      preferred_element_type=jnp.float32))
    x = _silu(jnp.dot(x, w2_ref[...], preferred_element_type=jnp.float32))
    w = jnp.dot(x, w3_ref[...], preferred_element_type=jnp.float32)
    s = lax.pad(sph_ref[...], jnp.float32(0), ((0, 0, 0), (0, C - 16, 0)))
    w_ref[...] = jnp.concatenate([w, s], axis=1)


def _pre_w(rb, sph_pad, W1, W2, W3):
    BE = 2000
    return pl.pallas_call(
        _pre_w_body,
        grid=(E // BE,),
        in_specs=[
            pl.BlockSpec((BE, R), lambda i: (i, 0)),
            pl.BlockSpec((BE, 16), lambda i: (i, 0)),
            pl.BlockSpec((R, H), lambda i: (0, 0)),
            pl.BlockSpec((H, H), lambda i: (0, 0)),
            pl.BlockSpec((H, NPATH * C), lambda i: (0, 0)),
        ],
        out_specs=pl.BlockSpec((BE, WROW), lambda i: (i, 0)),
        out_shape=jax.ShapeDtypeStruct((E, WROW), jnp.float32),
    )(rb, sph_pad, W1, W2, W3)


def _sc_edge_body(h_hbm, w_hbm, send_hbm, recv_hbm, out_hbm,
                  acc, rc0, sc0, rc1, sc1, m_nloc, m_send, m_edge,
                  h_st, w_st, semc0, semc1, semg):
    cid = lax.axis_index("c")
    sid = lax.axis_index("s")
    wid = sid * 2 + cid

    iota = lax.iota(jnp.int32, 16)
    zeros = jnp.zeros((16,), jnp.float32)
    izeros = jnp.zeros((16,), jnp.int32)

    def _init_m(i, _):
        m_nloc[pl.ds(i * 16, 16)] = izeros
        m_send[pl.ds(i * 16, 16)] = izeros
        m_edge[pl.ds(i * 16, 16)] = izeros
        return 0
    lax.fori_loop(0, MBUF // 16, _init_m, 0, unroll=False)

    def _process_batch(b, rem):
        cp_h = pltpu.async_copy(h_hbm.at[m_send.at[pl.ds(b * GB, GB)]],
                                h_st, semg)
        cp_w = pltpu.async_copy(w_hbm.at[m_edge.at[pl.ds(b * GB, GB)]],
                                w_st, semg)
        cp_h.wait()
        cp_w.wait()

        def _edge(j, _):
            jv = jnp.full((16,), j, jnp.int32)
            nloc = plsc.load_gather(
                m_nloc, [jnp.full((16,), b * GB + j, jnp.int32)])
            base = nloc * ROW
            s = [plsc.load_gather(
                    w_st, [jv, jnp.full((16,), NPATH * C + d, jnp.int32)])
                 for d in range(D)]
            for cc in range(C // 16):
                hn = plsc.load_gather(h_st, [jv, iota + cc * 16])
                t = []
                for l in range(NPATH):
                    wv = plsc.load_gather(
                        w_st, [jv, iota + (l * C + cc * 16)])
                    t.append(hn * wv)
                for d in range(D):
                    addr = base + (iota + (d * C + cc * 16))
                    plsc.addupdate_scatter(acc, [addr], t[_LIDX[d]] * s[d])
            return 0
        lax.fori_loop(0, rem, _edge, 0, unroll=False)

    for p in range(NPASS):
        node_lo = p * NODES_PER_PASS + wid * NODES_PER_TILE
        lo_v = jnp.full((16,), node_lo, jnp.int32)

        def _zero(i, _):
            acc[pl.ds(i * 16, 16)] = zeros
            return 0
        lax.fori_loop(0, (NODES_PER_TILE * ROW) // 16, _zero, 0,
                      unroll=False)

        def _scan_and_process(rc, sc, ebase, ptr):
            def _scan(v, pp):
                r = rc[pl.ds(v * 16, 16)]
                m = (r >= lo_v) & (r < lo_v + NODES_PER_TILE)
                cnt = plsc.all_reduce_population_count(m)[0]
                sv = sc[pl.ds(v * 16, 16)]
                ev = iota + (ebase + v * 16)
                plsc.store_compressed(
                    m_nloc.at[pl.ds(pp, 16)], r - lo_v, mask=m)
                plsc.store_compressed(
                    m_send.at[pl.ds(pp, 16)], sv, mask=m)
                plsc.store_compressed(
                    m_edge.at[pl.ds(pp, 16)], ev, mask=m)
                return pp + cnt

            ptr2 = lax.fori_loop(0, VPC, _scan, ptr, unroll=False)
            nfull = ptr2 // GB

            def _pb(b, _):
                _process_batch(b, GB)
                return 0
            lax.fori_loop(0, nfull, _pb, 0, unroll=False)

            q = nfull * GB
            for buf in (m_nloc, m_send, m_edge):
                v0 = buf[pl.ds(q, 16)]
                v1 = buf[pl.ds(q + 16, 16)]
                buf[pl.ds(0, 16)] = v0
                buf[pl.ds(16, 16)] = v1
            return ptr2 - q

        pltpu.async_copy(recv_hbm.at[pl.ds(0, CHUNK)], rc0, semc0)
        pltpu.async_copy(send_hbm.at[pl.ds(0, CHUNK)], sc0, semc0)

        def _cpair(ci2, ptr):
            e0 = (ci2 * 2) * CHUNK
            e1 = e0 + CHUNK
            pltpu.make_async_copy(
                recv_hbm.at[pl.ds(e0, CHUNK)], rc0, semc0).wait()
            pltpu.make_async_copy(
                send_hbm.at[pl.ds(e0, CHUNK)], sc0, semc0).wait()
            pltpu.async_copy(recv_hbm.at[pl.ds(e1, CHUNK)], rc1, semc1)
            pltpu.async_copy(send_hbm.at[pl.ds(e1, CHUNK)], sc1, semc1)
            ptr = _scan_and_process(rc0, sc0, e0, ptr)
            pltpu.make_async_copy(
                recv_hbm.at[pl.ds(e1, CHUNK)], rc1, semc1).wait()
            pltpu.make_async_copy(
                send_hbm.at[pl.ds(e1, CHUNK)], sc1, semc1).wait()
            e2 = jnp.minimum(e1 + CHUNK, E - CHUNK)
            pltpu.async_copy(recv_hbm.at[pl.ds(e2, CHUNK)], rc0, semc0)
            pltpu.async_copy(send_hbm.at[pl.ds(e2, CHUNK)], sc0, semc0)
            ptr = _scan_and_process(rc1, sc1, e1, ptr)
            return ptr

        ptr = lax.fori_loop(0, NCHUNK // 2, _cpair, 0, unroll=False)
        pltpu.make_async_copy(recv_hbm.at[pl.ds(0, CHUNK)], rc0, semc0).wait()
        pltpu.make_async_copy(send_hbm.at[pl.ds(0, CHUNK)], sc0, semc0).wait()

        @pl.when(ptr > 0)
        def _():
            _process_batch(0, ptr)

        pltpu.sync_copy(acc, out_hbm.at[pl.ds(node_lo * ROW,
                                              NODES_PER_TILE * ROW)])


def _sc_edge(h, w, send, recv):
    mesh = plsc.VectorSubcoreMesh(core_axis_name="c", subcore_axis_name="s")
    kfn = functools.partial(
        pl.kernel,
        mesh=mesh,
        compiler_params=pltpu.CompilerParams(needs_layout_passes=False),
        out_type=jax.ShapeDtypeStruct((N_PAD * ROW,), jnp.float32),
        scratch_types=[
            pltpu.VMEM((NODES_PER_TILE * ROW,), jnp.float32),
            pltpu.VMEM((CHUNK,), jnp.int32),
            pltpu.VMEM((CHUNK,), jnp.int32),
            pltpu.VMEM((CHUNK,), jnp.int32),
            pltpu.VMEM((CHUNK,), jnp.int32),
            pltpu.VMEM((MBUF,), jnp.int32),
            pltpu.VMEM((MBUF,), jnp.int32),
            pltpu.VMEM((MBUF,), jnp.int32),
            pltpu.VMEM((GB, C), jnp.float32),
            pltpu.VMEM((GB, WROW), jnp.float32),
            pltpu.SemaphoreType.DMA,
            pltpu.SemaphoreType.DMA,
            pltpu.SemaphoreType.DMA,
        ],
    )(_sc_edge_body)
    return kfn(h, w, send, recv)


def _post_body(agg_ref, attr_ref, wp_ref, wm_ref, out_ref):
    BN = agg_ref.shape[0]
    agg = agg_ref[...].reshape(BN, D, C)
    attr = attr_ref[...]
    outs = []
    off = 0
    for li, d in enumerate(LDIMS):
        a_l = agg[:, off:off + d, :].reshape(BN * d, C)
        p_l = jnp.dot(a_l, wp_ref[li],
                      preferred_element_type=jnp.float32)
        g_l = jnp.dot(p_l, wm_ref[li],
                      preferred_element_type=jnp.float32).reshape(BN, d, A, C)
        o_l = jnp.zeros((BN, d, C), jnp.float32)
        for a in range(A):
            o_l = o_l + attr[:, a][:, None, None] * g_l[:, :, a, :]
        outs.append(jnp.transpose(o_l, (0, 2, 1)))
        off += d
    out_ref[...] = jnp.concatenate(outs, axis=2)


def _post(agg, attr_pad, W_post, Wmix_r):
    BN = 64
    grid = N_PAD // BN
    return pl.pallas_call(
        _post_body,
        grid=(grid,),
        in_specs=[
            pl.BlockSpec((BN, ROW), lambda i: (i, 0)),
            pl.BlockSpec((BN, A), lambda i: (i, 0)),
            pl.BlockSpec((NPATH, C, C), lambda i: (0, 0, 0)),
            pl.BlockSpec((NPATH, C, A * C), lambda i: (0, 0, 0)),
        ],
        out_specs=pl.BlockSpec((BN, C, D), lambda i: (i, 0, 0)),
        out_shape=jax.ShapeDtypeStruct((N_PAD, C, D), jnp.float32),
    )(agg, attr_pad, W_post, Wmix_r)


def kernel(node_features, node_attributes, sph_harmonics, radial_basis,
           edge_index, W_pre, W1, W2, W3, W_post, W_mix):
    send = edge_index[0].astype(jnp.int32)
    recv = edge_index[1].astype(jnp.int32)
    sph_pad = jnp.pad(sph_harmonics, ((0, 0), (0, 16 - D)))

    h = _pre_h(node_features, W_pre)
    w = _pre_w(radial_basis, sph_pad, W1, W2, W3)
    agg_flat = _sc_edge(h, w, send, recv)
    agg = agg_flat.reshape(N_PAD, ROW)

    attr_pad = jnp.pad(node_attributes, ((0, N_PAD - N), (0, 0)))
    Wmix_r = W_mix.reshape(NPATH, C, A * C)
    out = _post(agg, attr_pad, W_post, Wmix_r)
    return out[:N]


# speedup vs baseline: 1.3157x; 1.3157x over previous
import functools
import jax
import jax.numpy as jnp
from jax import lax
from jax.experimental import pallas as pl
from jax.experimental.pallas import tpu as pltpu
from jax.experimental.pallas import tpu_sc as plsc

N = 10000
E = 160000
C = 128
A = 10
R = 8
LDIMS = [1, 3, 5]
D = 9
NPATH = 3
H = 64

NUM_TILES = 32
NODES_PER_TILE = 80
NODES_PER_PASS = NUM_TILES * NODES_PER_TILE
NPASS = 4
N_PAD = NPASS * NODES_PER_PASS

CHUNK = 800
NCHUNK = E // CHUNK
VPC = CHUNK // 16
MBUF = CHUNK + 64
GB = 32
ROW = D * C
WROW = (NPATH + 1) * C

_LIDX = []
for _li, _d in enumerate(LDIMS):
    _LIDX += [_li] * _d


def _silu(x):
    return x * jax.nn.sigmoid(x)


def _pre_h_body(nf_ref, wp_ref, h_ref):
    h_ref[...] = jnp.dot(nf_ref[...], wp_ref[...],
                         preferred_element_type=jnp.float32)


def _pre_h(nf, W_pre):
    BN = 1000
    return pl.pallas_call(
        _pre_h_body,
        grid=(N // BN,),
        in_specs=[
            pl.BlockSpec((BN, C), lambda i: (i, 0)),
            pl.BlockSpec((C, C), lambda i: (0, 0)),
        ],
        out_specs=pl.BlockSpec((BN, C), lambda i: (i, 0)),
        out_shape=jax.ShapeDtypeStruct((N, C), jnp.float32),
    )(nf, W_pre)


def _pre_w_body(rb_ref, sph_ref, w1_ref, w2_ref, w3_ref, w_ref):
    x = _silu(jnp.dot(rb_ref[...], w1_ref[...],
                      preferred_element_type=jnp.float32))
    x = _silu(jnp.dot(x, w2_ref[...], preferred_element_type=jnp.float32))
    w = jnp.dot(x, w3_ref[...], preferred_element_type=jnp.float32)
    s = lax.pad(sph_ref[...], jnp.float32(0), ((0, 0, 0), (0, C - D, 0)))
    w_ref[...] = jnp.concatenate([w, s], axis=1)


def _pre_w(rb, sph, W1, W2, W3):
    BE = 2000
    return pl.pallas_call(
        _pre_w_body,
        grid=(E // BE,),
        in_specs=[
            pl.BlockSpec((BE, R), lambda i: (i, 0)),
            pl.BlockSpec((BE, D), lambda i: (i, 0)),
            pl.BlockSpec((R, H), lambda i: (0, 0)),
            pl.BlockSpec((H, H), lambda i: (0, 0)),
            pl.BlockSpec((H, NPATH * C), lambda i: (0, 0)),
        ],
        out_specs=pl.BlockSpec((BE, WROW), lambda i: (i, 0)),
        out_shape=jax.ShapeDtypeStruct((E, WROW), jnp.float32),
    )(rb, sph, W1, W2, W3)


def _sc_edge_body(h_hbm, w_hbm, send_hbm, recv_hbm, out_hbm,
                  acc, rc0, sc0, rc1, sc1, m_nloc, m_send, m_edge,
                  h_st, w_st, semc0, semc1, semg):
    cid = lax.axis_index("c")
    sid = lax.axis_index("s")
    wid = sid * 2 + cid

    iota = lax.iota(jnp.int32, 16)
    zeros = jnp.zeros((16,), jnp.float32)
    izeros = jnp.zeros((16,), jnp.int32)

    def _init_m(i, _):
        m_nloc[pl.ds(i * 16, 16)] = izeros
        m_send[pl.ds(i * 16, 16)] = izeros
        m_edge[pl.ds(i * 16, 16)] = izeros
        return 0
    lax.fori_loop(0, MBUF // 16, _init_m, 0, unroll=False)

    def _process_batch(b, rem):
        cp_h = pltpu.async_copy(h_hbm.at[m_send.at[pl.ds(b * GB, GB)]],
                                h_st, semg)
        cp_w = pltpu.async_copy(w_hbm.at[m_edge.at[pl.ds(b * GB, GB)]],
                                w_st, semg)
        cp_h.wait()
        cp_w.wait()

        def _edge(j, _):
            jv = jnp.full((16,), j, jnp.int32)
            nv = m_nloc[pl.ds(b * GB + j, 16)]
            base = nv[0] * ROW
            s = [plsc.load_gather(
                    w_st, [jv, jnp.full((16,), NPATH * C + d, jnp.int32)])
                 for d in range(D)]
            for cc in range(C // 16):
                hn = h_st[j, pl.ds(cc * 16, 16)]
                t = []
                for l in range(NPATH):
                    t.append(hn * w_st[j, pl.ds(l * C + cc * 16, 16)])
                for d in range(D):
                    plsc.addupdate(
                        acc.at[pl.ds(base + (d * C + cc * 16), 16)],
                        t[_LIDX[d]] * s[d])
            return 0
        lax.fori_loop(0, rem, _edge, 0, unroll=False)

    for p in range(NPASS):
        node_lo = p * NODES_PER_PASS + wid * NODES_PER_TILE
        lo_v = jnp.full((16,), node_lo, jnp.int32)

        def _zero(i, _):
            acc[pl.ds(i * 16, 16)] = zeros
            return 0
        lax.fori_loop(0, (NODES_PER_TILE * ROW) // 16, _zero, 0,
                      unroll=False)

        def _scan_and_process(rc, sc, ebase, ptr):
            def _scan(v, pp):
                r = rc[pl.ds(v * 16, 16)]
                m = (r >= lo_v) & (r < lo_v + NODES_PER_TILE)
                cnt = plsc.all_reduce_population_count(m)[0]
                sv = sc[pl.ds(v * 16, 16)]
                ev = iota + (ebase + v * 16)
                plsc.store_compressed(
                    m_nloc.at[pl.ds(pp, 16)], r - lo_v, mask=m)
                plsc.store_compressed(
                    m_send.at[pl.ds(pp, 16)], sv, mask=m)
                plsc.store_compressed(
                    m_edge.at[pl.ds(pp, 16)], ev, mask=m)
                return pp + cnt

            ptr2 = lax.fori_loop(0, VPC, _scan, ptr, unroll=2)
            nfull = ptr2 // GB

            def _pb(b, _):
                _process_batch(b, GB)
                return 0
            lax.fori_loop(0, nfull, _pb, 0, unroll=False)

            q = nfull * GB
            for buf in (m_nloc, m_send, m_edge):
                v0 = buf[pl.ds(q, 16)]
                v1 = buf[pl.ds(q + 16, 16)]
                buf[pl.ds(0, 16)] = v0
                buf[pl.ds(16, 16)] = v1
            return ptr2 - q

        pltpu.async_copy(recv_hbm.at[pl.ds(0, CHUNK)], rc0, semc0)
        pltpu.async_copy(send_hbm.at[pl.ds(0, CHUNK)], sc0, semc0)

        def _cpair(ci2, ptr):
            e0 = (ci2 * 2) * CHUNK
            e1 = e0 + CHUNK
            pltpu.make_async_copy(
                recv_hbm.at[pl.ds(e0, CHUNK)], rc0, semc0).wait()
            pltpu.make_async_copy(
                send_hbm.at[pl.ds(e0, CHUNK)], sc0, semc0).wait()
            pltpu.async_copy(recv_hbm.at[pl.ds(e1, CHUNK)], rc1, semc1)
            pltpu.async_copy(send_hbm.at[pl.ds(e1, CHUNK)], sc1, semc1)
            ptr = _scan_and_process(rc0, sc0, e0, ptr)
            pltpu.make_async_copy(
                recv_hbm.at[pl.ds(e1, CHUNK)], rc1, semc1).wait()
            pltpu.make_async_copy(
                send_hbm.at[pl.ds(e1, CHUNK)], sc1, semc1).wait()
            e2 = jnp.minimum(e1 + CHUNK, E - CHUNK)
            pltpu.async_copy(recv_hbm.at[pl.ds(e2, CHUNK)], rc0, semc0)
            pltpu.async_copy(send_hbm.at[pl.ds(e2, CHUNK)], sc0, semc0)
            ptr = _scan_and_process(rc1, sc1, e1, ptr)
            return ptr

        ptr = lax.fori_loop(0, NCHUNK // 2, _cpair, 0, unroll=False)
        pltpu.make_async_copy(recv_hbm.at[pl.ds(0, CHUNK)], rc0, semc0).wait()
        pltpu.make_async_copy(send_hbm.at[pl.ds(0, CHUNK)], sc0, semc0).wait()

        @pl.when(ptr > 0)
        def _():
            _process_batch(0, ptr)

        pltpu.sync_copy(acc, out_hbm.at[pl.ds(node_lo * ROW,
                                              NODES_PER_TILE * ROW)])


def _sc_edge(h, w, send, recv):
    mesh = plsc.VectorSubcoreMesh(core_axis_name="c", subcore_axis_name="s")
    kfn = functools.partial(
        pl.kernel,
        mesh=mesh,
        compiler_params=pltpu.CompilerParams(needs_layout_passes=False),
        out_type=jax.ShapeDtypeStruct((N_PAD * ROW,), jnp.float32),
        scratch_types=[
            pltpu.VMEM((NODES_PER_TILE * ROW,), jnp.float32),
            pltpu.VMEM((CHUNK,), jnp.int32),
            pltpu.VMEM((CHUNK,), jnp.int32),
            pltpu.VMEM((CHUNK,), jnp.int32),
            pltpu.VMEM((CHUNK,), jnp.int32),
            pltpu.VMEM((MBUF,), jnp.int32),
            pltpu.VMEM((MBUF,), jnp.int32),
            pltpu.VMEM((MBUF,), jnp.int32),
            pltpu.VMEM((GB, C), jnp.float32),
            pltpu.VMEM((GB, WROW), jnp.float32),
            pltpu.SemaphoreType.DMA,
            pltpu.SemaphoreType.DMA,
            pltpu.SemaphoreType.DMA,
        ],
    )(_sc_edge_body)
    return kfn(h, w, send, recv)


def _post_body(agg_ref, attr_ref, wp_ref, wm_ref, out_ref):
    BN = agg_ref.shape[0]
    agg = agg_ref[...].reshape(BN, D, C)
    attr = attr_ref[...]
    outs = []
    off = 0
    for li, d in enumerate(LDIMS):
        a_l = agg[:, off:off + d, :].reshape(BN * d, C)
        p_l = jnp.dot(a_l, wp_ref[li],
                      preferred_element_type=jnp.float32).reshape(BN, d, C)
        m_l = jnp.dot(attr, wm_ref[li],
                      preferred_element_type=jnp.float32).reshape(BN, C, C)
        o_l = lax.dot_general(
            m_l, p_l,
            dimension_numbers=(((2,), (2,)), ((0,), (0,))),
            preferred_element_type=jnp.float32)
        outs.append(o_l)
        off += d
    out_ref[...] = jnp.concatenate(outs, axis=2)


def _post(agg, attr, W_post, Wmix_r):
    BN = 80
    grid = N // BN
    return pl.pallas_call(
        _post_body,
        grid=(grid,),
        in_specs=[
            pl.BlockSpec((BN, ROW), lambda i: (i, 0)),
            pl.BlockSpec((BN, A), lambda i: (i, 0)),
            pl.BlockSpec((NPATH, C, C), lambda i: (0, 0, 0)),
            pl.BlockSpec((NPATH, A, C * C), lambda i: (0, 0, 0)),
        ],
        out_specs=pl.BlockSpec((BN, C, D), lambda i: (i, 0, 0)),
        out_shape=jax.ShapeDtypeStruct((N, C, D), jnp.float32),
    )(agg, attr, W_post, Wmix_r)


def kernel(node_features, node_attributes, sph_harmonics, radial_basis,
           edge_index, W_pre, W1, W2, W3, W_post, W_mix):
    send = edge_index[0].astype(jnp.int32)
    recv = edge_index[1].astype(jnp.int32)

    h = _pre_h(node_features, W_pre)
    w = _pre_w(radial_basis, sph_harmonics, W1, W2, W3)
    agg_flat = _sc_edge(h, w, send, recv)
    agg = agg_flat.reshape(N_PAD, ROW)

    Wmix_t = jnp.transpose(W_mix, (0, 2, 3, 1)).reshape(NPATH, A, C * C)
    return _post(agg, node_attributes, W_post, Wmix_t)
